# baseline (speedup 1.0000x reference)
import functools

import jax
import jax.numpy as jnp
from jax import lax
from jax.experimental import pallas as pl
from jax.experimental.pallas import tpu as pltpu
from jax.experimental.pallas import tpu_sc as plsc

NC = 2
NS = 16
NW = NC * NS
LN = 16
GRP = 128
GB = 8


def _tc_project(x, W, asv, adv):
    N, D = x.shape
    C = W.shape[1]

    def body(x_ref, w_ref, as_ref, ad_ref, h_ref, ab_ref):
        h = jnp.dot(x_ref[...], w_ref[...], preferred_element_type=jnp.float32)
        h_ref[...] = h
        a = jnp.sum(h * as_ref[...], axis=1)
        b = jnp.sum(h * ad_ref[...], axis=1)
        ab_ref[...] = jnp.concatenate([a[None, None, :], b[None, None, :]], axis=0)

    return pl.pallas_call(
        body,
        out_shape=[
            jax.ShapeDtypeStruct((N, C), jnp.float32),
            jax.ShapeDtypeStruct((2, 1, N), jnp.float32),
        ],
    )(x, W, asv, adv)


def _sc_edge_pass(srcg, dstg, ewg, ab, nm):
    GP = srcg.shape[0]
    GPT = GP // NW
    NV = nm.shape[0]
    mesh = plsc.VectorSubcoreMesh(core_axis_name="c", subcore_axis_name="s",
                                  num_cores=NC, num_subcores=NS)

    @functools.partial(
        pl.kernel,
        out_type=[
            jax.ShapeDtypeStruct((NC, 1, NV), jnp.float32),
            jax.ShapeDtypeStruct((GP, GRP), jnp.float32),
        ],
        mesh=mesh,
        scratch_types=[
            pltpu.VMEM((NV,), jnp.float32),
            pltpu.VMEM((NV,), jnp.float32),
            pltpu.VMEM((NV,), jnp.int32),
            pltpu.VMEM((GPT, GRP), jnp.int32),
            pltpu.VMEM((GPT, GRP), jnp.int32),
            pltpu.VMEM((GPT, GRP), jnp.float32),
            pltpu.VMEM((GPT, GRP), jnp.float32),
            pltpu.VMEM((GPT, GRP), jnp.float32),
            pltpu.VMEM_SHARED((NV,), jnp.float32),
        ],
        compiler_params=pltpu.CompilerParams(needs_layout_passes=False),
    )
    def k(src_hbm, dst_hbm, ew_hbm, ab_hbm, nm_hbm,
          pden_hbm, pw_hbm,
          a_t, b_t, nm_t, sg, dg, eg, p_c, pw_c, den_sh):
        c = lax.axis_index("c")
        s = lax.axis_index("s")
        w = c * NS + s

        @pl.when(s == 0)
        def _zero():
            def zb(i, carry):
                a_t[pl.ds(i * LN, LN)] = jnp.zeros((LN,), jnp.float32)
                return carry
            lax.fori_loop(0, NV // LN, zb, 0)
            pltpu.sync_copy(a_t, den_sh)

        plsc.subcore_barrier()

        pltpu.sync_copy(ab_hbm.at[0, 0], a_t)
        pltpu.sync_copy(ab_hbm.at[1, 0], b_t)
        pltpu.sync_copy(nm_hbm, nm_t)
        base = w * GPT
        pltpu.sync_copy(src_hbm.at[pl.ds(base, GPT)], sg)
        pltpu.sync_copy(dst_hbm.at[pl.ds(base, GPT)], dg)
        pltpu.sync_copy(ew_hbm.at[pl.ds(base, GPT)], eg)

        def gbody(g, carry):
            for l in range(GRP // LN):
                sl = pl.ds(l * LN, LN)
                isv = sg[g, sl]
                idv = dg[g, sl]
                ewv = eg[g, sl]
                av = plsc.load_gather(a_t, [isv])
                bv = plsc.load_gather(b_t, [idv])
                ms = plsc.load_gather(nm_t, [isv])
                md = plsc.load_gather(nm_t, [idv])
                e = av + bv
                e = jnp.where(e >= 0.0, e, 0.2 * e)
                valid = (ms == md) & (ewv != 0.0)
                p = jnp.where(valid, jnp.exp(e), 0.0)
                p_c[g, sl] = p
                pw_c[g, sl] = p * ewv
            pltpu.sync_copy(p_c.at[g], den_sh.at[dg.at[g]], add=True)
            return carry
        lax.fori_loop(0, GPT, gbody, 0)

        pltpu.sync_copy(pw_c, pw_hbm.at[pl.ds(base, GPT)])

        plsc.subcore_barrier()

        @pl.when(s == 0)
        def _flush():
            pltpu.sync_copy(den_sh, pden_hbm.at[c, 0])

    return k(srcg, dstg, ewg, ab, nm)


def _sc_msg_pass(srcg, dstg, pwg, pden, h):
    GP = srcg.shape[0]
    GPT = GP // NW
    NV, C = h.shape
    ZB = 16
    NBT = (NV // ZB + NS - 1) // NS
    mesh = plsc.VectorSubcoreMesh(core_axis_name="c", subcore_axis_name="s",
                                  num_cores=NC, num_subcores=NS)

    GBK = 16
    DCH = 1024
    GPT0 = int(round(0.3 * 2 * GPT / GBK)) * GBK
    GPT1 = 2 * GPT - GPT0
    NBLK0 = GPT0 // GBK
    NBLK1 = GPT1 // GBK

    @functools.partial(
        pl.kernel,
        out_type=jax.ShapeDtypeStruct((NC, NV, C), jnp.float32),
        mesh=mesh,
        scratch_types=[
            pltpu.VMEM((NV,), jnp.float32),
            pltpu.VMEM((DCH,), jnp.float32),
            pltpu.VMEM((2, GRP, C), jnp.float32),
            pltpu.VMEM((GBK, GRP), jnp.int32),
            pltpu.VMEM((GBK, GRP), jnp.int32),
            pltpu.VMEM((GBK, GRP), jnp.float32),
            pltpu.VMEM((GRP,), jnp.int32),
            pltpu.VMEM_SHARED((NV + 8, C), jnp.float32),
            pltpu.SemaphoreType.DMA,
            pltpu.SemaphoreType.DMA,
            pltpu.SemaphoreType.DMA,
            pltpu.SemaphoreType.DMA,
        ],
        compiler_params=pltpu.CompilerParams(needs_layout_passes=False),
    )
    def k(src_hbm, dst_hbm, pw_hbm, pden_hbm, h_hbm,
          out_hbm,
          den_t, tmp, rows2, sgb, dgb, pwb, tidx, out_sh,
          gsem0, gsem1, ssem0, ssem1):
        c = lax.axis_index("c")
        s = lax.axis_index("s")
        w = c * NS + s
        gsem = (gsem0, gsem1)
        ssem = (ssem0, ssem1)

        def zr(i, carry):
            for b in range(2):
                for j in range(C // LN):
                    rows2[b, i, pl.ds(j * LN, LN)] = jnp.zeros((LN,), jnp.float32)
            return carry
        lax.fori_loop(0, GRP, zr, 0)
        for l in range(GRP // LN):
            tidx[pl.ds(l * LN, LN)] = jnp.full((LN,), NV, jnp.int32)

        def zo(nb, carry):
            row0 = (s * NBT + nb) * ZB
            @pl.when(row0 < NV)
            def _():
                pltpu.sync_copy(rows2.at[0].at[pl.ds(0, ZB)],
                                out_sh.at[pl.ds(row0, ZB)])
            return carry
        lax.fori_loop(0, NBT, zo, 0)
        plsc.subcore_barrier()

        pltpu.sync_copy(pden_hbm.at[0, 0], den_t)
        off = 0
        while off < NV:
            sz = min(DCH, NV - off)
            pltpu.sync_copy(pden_hbm.at[1, 0, pl.ds(off, sz)],
                            tmp.at[pl.ds(0, sz)])
            def db(i, carry, off=off):
                sl = pl.ds(off + i * LN, LN)
                v = den_t[sl] + tmp[pl.ds(i * LN, LN)]
                den_t[sl] = jnp.where(v == 0.0, 1.0, v)
                return carry
            lax.fori_loop(0, sz // LN, db, 0)
            off += sz

        def start_gather(g, b):
            pltpu.async_copy(h_hbm.at[sgb.at[g]], rows2.at[b], gsem[b])

        def wait_gather(b):
            pltpu.make_async_copy(h_hbm.at[sgb.at[0]], rows2.at[b],
                                  gsem[b]).wait()

        def start_scatter(g, b):
            pltpu.async_copy(rows2.at[b], out_sh.at[dgb.at[g]], ssem[b],
                             add=True)

        def start_scatter_trash():
            pltpu.async_copy(rows2.at[1], out_sh.at[tidx], ssem[1], add=True)

        def wait_scatter(b):
            pltpu.make_async_copy(rows2.at[b], out_sh.at[dgb.at[0]],
                                  ssem[b]).wait()

        def scale(g, b):
            def rbody(l, rc):
                sl = pl.ds(l * LN, LN)
                dv = plsc.load_gather(den_t, [dgb[g, sl]])
                cv = pwb[g, sl] / dv
                for ii in range(LN):
                    i = l * LN + ii
                    cs = jnp.full((LN,), cv[ii], jnp.float32)
                    for jj in range(C // LN):
                        sj = pl.ds(jj * LN, LN)
                        rows2[b, i, sj] = rows2[b, i, sj] * cs
                return rc
            lax.fori_loop(0, GRP // LN, rbody, 0)

        gbase = c * (NS * GPT0) + s * jnp.where(c == 0, GPT0, GPT1)
        nblk = jnp.where(c == 0, NBLK0, NBLK1)

        def blk(nb, carry):
            base = gbase + nb * GBK
            wait_scatter(1)
            pltpu.sync_copy(src_hbm.at[pl.ds(base, GBK)], sgb)
            pltpu.sync_copy(dst_hbm.at[pl.ds(base, GBK)], dgb)
            pltpu.sync_copy(pw_hbm.at[pl.ds(base, GBK)], pwb)
            start_gather(0, 0)
            start_scatter_trash()

            def pbody(i, pc):
                gA = 2 * i
                gB = 2 * i + 1
                wait_gather(0)
                scale(gA, 0)
                wait_scatter(1)
                start_gather(gB, 1)
                start_scatter(gA, 0)
                wait_gather(1)
                scale(gB, 1)
                wait_scatter(0)
                start_gather(jnp.minimum(gB + 1, GBK - 1), 0)
                start_scatter(gB, 1)
                return pc
            lax.fori_loop(0, GBK // 2, pbody, 0)
            wait_gather(0)
            return carry

        start_scatter_trash()
        lax.fori_loop(0, nblk, blk, 0)
        wait_scatter(1)

        plsc.subcore_barrier()

        @pl.when(s == 0)
        def _flush():
            pltpu.sync_copy(out_sh.at[pl.ds(0, NV)], out_hbm.at[c])

    return k(srcg, dstg, pwg, pden, h)


def _tc_finish(outp, bias2):
    _, N, C = outp.shape

    def body(o_ref, b_ref, out_ref):
        out_ref[...] = o_ref[0] + o_ref[1] + b_ref[...]

    return pl.pallas_call(
        body,
        out_shape=jax.ShapeDtypeStruct((N, C), jnp.float32),
    )(outp, bias2)


def kernel(x, edge_index, edge_weight, modality_groups, W, att_src, att_dst, bias):
    N, D = x.shape
    C = W.shape[1]
    E = edge_index.shape[1]

    nm = jnp.zeros((N,), jnp.int32)
    for g in range(modality_groups.shape[0]):
        nm = nm.at[modality_groups[g]].set(g)

    QUANT = NW * GRP * GB
    EP = ((E + QUANT - 1) // QUANT) * QUANT
    pad = EP - E
    srcg = jnp.concatenate([edge_index[0], jnp.zeros((pad,), jnp.int32)]).reshape(EP // GRP, GRP)
    dstg = jnp.concatenate([edge_index[1], jnp.zeros((pad,), jnp.int32)]).reshape(EP // GRP, GRP)
    ewg = jnp.concatenate([edge_weight, jnp.zeros((pad,), jnp.float32)]).reshape(EP // GRP, GRP)

    asv = att_src.reshape(1, C).astype(jnp.float32)
    adv = att_dst.reshape(1, C).astype(jnp.float32)

    h, ab = _tc_project(x, W, asv, adv)
    pden, pwg = _sc_edge_pass(srcg, dstg, ewg, ab, nm)
    outp = _sc_msg_pass(srcg, dstg, pwg, pden, h)
    return _tc_finish(outp, bias.reshape(1, C))


# speedup vs baseline: 1.1820x; 1.1820x over previous
import functools

import jax
import jax.numpy as jnp
from jax import lax
from jax.experimental import pallas as pl
from jax.experimental.pallas import tpu as pltpu
from jax.experimental.pallas import tpu_sc as plsc

NC = 2
NS = 16
NW = NC * NS
LN = 16
GRP = 128
GB = 8


def _tc_project(x, W, asv, adv):
    N, D = x.shape
    C = W.shape[1]

    def body(x_ref, w_ref, as_ref, ad_ref, h_ref, ab_ref):
        h = jnp.dot(x_ref[...], w_ref[...], preferred_element_type=jnp.float32)
        h_ref[...] = h
        a = jnp.sum(h * as_ref[...], axis=1)
        b = jnp.sum(h * ad_ref[...], axis=1)
        ab_ref[...] = jnp.concatenate([a[None, None, :], b[None, None, :]], axis=0)

    return pl.pallas_call(
        body,
        out_shape=[
            jax.ShapeDtypeStruct((N, C), jnp.float32),
            jax.ShapeDtypeStruct((2, 1, N), jnp.float32),
        ],
    )(x, W, asv, adv)


def _sc_edge_pass(srcg, dstg, ewg, ab, nm):
    GP = srcg.shape[0]
    GPT = GP // NW
    NV = nm.shape[0]
    mesh = plsc.VectorSubcoreMesh(core_axis_name="c", subcore_axis_name="s",
                                  num_cores=NC, num_subcores=NS)

    @functools.partial(
        pl.kernel,
        out_type=[
            jax.ShapeDtypeStruct((NC, 1, NV), jnp.float32),
            jax.ShapeDtypeStruct((GP, GRP), jnp.float32),
        ],
        mesh=mesh,
        scratch_types=[
            pltpu.VMEM((NV,), jnp.float32),
            pltpu.VMEM((NV,), jnp.float32),
            pltpu.VMEM((NV,), jnp.int32),
            pltpu.VMEM((GPT, GRP), jnp.int32),
            pltpu.VMEM((GPT, GRP), jnp.int32),
            pltpu.VMEM((GPT, GRP), jnp.float32),
            pltpu.VMEM((GPT, GRP), jnp.float32),
            pltpu.VMEM((GPT, GRP), jnp.float32),
            pltpu.VMEM_SHARED((NV,), jnp.float32),
        ],
        compiler_params=pltpu.CompilerParams(needs_layout_passes=False),
    )
    def k(src_hbm, dst_hbm, ew_hbm, ab_hbm, nm_hbm,
          pden_hbm, pw_hbm,
          a_t, b_t, nm_t, sg, dg, eg, p_c, pw_c, den_sh):
        c = lax.axis_index("c")
        s = lax.axis_index("s")
        w = c * NS + s

        @pl.when(s == 0)
        def _zero():
            def zb(i, carry):
                a_t[pl.ds(i * LN, LN)] = jnp.zeros((LN,), jnp.float32)
                return carry
            lax.fori_loop(0, NV // LN, zb, 0)
            pltpu.sync_copy(a_t, den_sh)

        plsc.subcore_barrier()

        pltpu.sync_copy(ab_hbm.at[0, 0], a_t)
        pltpu.sync_copy(ab_hbm.at[1, 0], b_t)
        pltpu.sync_copy(nm_hbm, nm_t)
        base = w * GPT
        pltpu.sync_copy(src_hbm.at[pl.ds(base, GPT)], sg)
        pltpu.sync_copy(dst_hbm.at[pl.ds(base, GPT)], dg)
        pltpu.sync_copy(ew_hbm.at[pl.ds(base, GPT)], eg)

        def gbody(g, carry):
            for l in range(GRP // LN):
                sl = pl.ds(l * LN, LN)
                isv = sg[g, sl]
                idv = dg[g, sl]
                ewv = eg[g, sl]
                av = plsc.load_gather(a_t, [isv])
                bv = plsc.load_gather(b_t, [idv])
                ms = plsc.load_gather(nm_t, [isv])
                md = plsc.load_gather(nm_t, [idv])
                e = av + bv
                e = jnp.where(e >= 0.0, e, 0.2 * e)
                valid = (ms == md) & (ewv != 0.0)
                p = jnp.where(valid, jnp.exp(e), 0.0)
                p_c[g, sl] = p
                pw_c[g, sl] = p * ewv
            pltpu.sync_copy(p_c.at[g], den_sh.at[dg.at[g]], add=True)
            return carry
        lax.fori_loop(0, GPT, gbody, 0)

        pltpu.sync_copy(pw_c, pw_hbm.at[pl.ds(base, GPT)])

        plsc.subcore_barrier()

        @pl.when(s == 0)
        def _flush():
            pltpu.sync_copy(den_sh, pden_hbm.at[c, 0])

    return k(srcg, dstg, ewg, ab, nm)


def _sc_msg_pass(srcg, dstg, pwg, pden, h):
    GP = srcg.shape[0]
    GPT = GP // NW
    NV, C = h.shape
    ZB = 16
    NBT = (NV // ZB + NS - 1) // NS
    mesh = plsc.VectorSubcoreMesh(core_axis_name="c", subcore_axis_name="s",
                                  num_cores=NC, num_subcores=NS)

    GBK = 16
    DCH = 1024
    GPT0 = int(round(0.7 * 2 * GPT / GBK)) * GBK
    GPT1 = 2 * GPT - GPT0
    NBLK0 = GPT0 // GBK
    NBLK1 = GPT1 // GBK

    @functools.partial(
        pl.kernel,
        out_type=jax.ShapeDtypeStruct((NC, NV, C), jnp.float32),
        mesh=mesh,
        scratch_types=[
            pltpu.VMEM((NV,), jnp.float32),
            pltpu.VMEM((DCH,), jnp.float32),
            pltpu.VMEM((2, GRP, C), jnp.float32),
            pltpu.VMEM((GBK, GRP), jnp.int32),
            pltpu.VMEM((GBK, GRP), jnp.int32),
            pltpu.VMEM((GBK, GRP), jnp.float32),
            pltpu.VMEM((GRP,), jnp.int32),
            pltpu.VMEM_SHARED((NV + 8, C), jnp.float32),
            pltpu.SemaphoreType.DMA,
            pltpu.SemaphoreType.DMA,
            pltpu.SemaphoreType.DMA,
            pltpu.SemaphoreType.DMA,
        ],
        compiler_params=pltpu.CompilerParams(needs_layout_passes=False),
    )
    def k(src_hbm, dst_hbm, pw_hbm, pden_hbm, h_hbm,
          out_hbm,
          den_t, tmp, rows2, sgb, dgb, pwb, tidx, out_sh,
          gsem0, gsem1, ssem0, ssem1):
        c = lax.axis_index("c")
        s = lax.axis_index("s")
        w = c * NS + s
        gsem = (gsem0, gsem1)
        ssem = (ssem0, ssem1)

        def zr(i, carry):
            for b in range(2):
                for j in range(C // LN):
                    rows2[b, i, pl.ds(j * LN, LN)] = jnp.zeros((LN,), jnp.float32)
            return carry
        lax.fori_loop(0, GRP, zr, 0)
        for l in range(GRP // LN):
            tidx[pl.ds(l * LN, LN)] = jnp.full((LN,), NV, jnp.int32)

        def zo(nb, carry):
            row0 = (s * NBT + nb) * ZB
            @pl.when(row0 < NV)
            def _():
                pltpu.sync_copy(rows2.at[0].at[pl.ds(0, ZB)],
                                out_sh.at[pl.ds(row0, ZB)])
            return carry
        lax.fori_loop(0, NBT, zo, 0)
        plsc.subcore_barrier()

        pltpu.sync_copy(pden_hbm.at[0, 0], den_t)
        off = 0
        while off < NV:
            sz = min(DCH, NV - off)
            pltpu.sync_copy(pden_hbm.at[1, 0, pl.ds(off, sz)],
                            tmp.at[pl.ds(0, sz)])
            def db(i, carry, off=off):
                sl = pl.ds(off + i * LN, LN)
                v = den_t[sl] + tmp[pl.ds(i * LN, LN)]
                den_t[sl] = jnp.where(v == 0.0, 1.0, v)
                return carry
            lax.fori_loop(0, sz // LN, db, 0)
            off += sz

        def start_gather(g, b):
            pltpu.async_copy(h_hbm.at[sgb.at[g]], rows2.at[b], gsem[b])

        def wait_gather(b):
            pltpu.make_async_copy(h_hbm.at[sgb.at[0]], rows2.at[b],
                                  gsem[b]).wait()

        def start_scatter(g, b):
            pltpu.async_copy(rows2.at[b], out_sh.at[dgb.at[g]], ssem[b],
                             add=True)

        def start_scatter_trash():
            pltpu.async_copy(rows2.at[1], out_sh.at[tidx], ssem[1], add=True)

        def wait_scatter(b):
            pltpu.make_async_copy(rows2.at[b], out_sh.at[dgb.at[0]],
                                  ssem[b]).wait()

        def scale(g, b):
            def rbody(l, rc):
                sl = pl.ds(l * LN, LN)
                dv = plsc.load_gather(den_t, [dgb[g, sl]])
                cv = pwb[g, sl] / dv
                for ii in range(LN):
                    i = l * LN + ii
                    cs = jnp.full((LN,), cv[ii], jnp.float32)
                    for jj in range(C // LN):
                        sj = pl.ds(jj * LN, LN)
                        rows2[b, i, sj] = rows2[b, i, sj] * cs
                return rc
            lax.fori_loop(0, GRP // LN, rbody, 0)

        gbase = c * (NS * GPT0) + s * jnp.where(c == 0, GPT0, GPT1)
        nblk = jnp.where(c == 0, NBLK0, NBLK1)

        def blk(nb, carry):
            base = gbase + nb * GBK
            wait_scatter(1)
            pltpu.sync_copy(src_hbm.at[pl.ds(base, GBK)], sgb)
            pltpu.sync_copy(dst_hbm.at[pl.ds(base, GBK)], dgb)
            pltpu.sync_copy(pw_hbm.at[pl.ds(base, GBK)], pwb)
            start_gather(0, 0)
            start_scatter_trash()

            def pbody(i, pc):
                gA = 2 * i
                gB = 2 * i + 1
                wait_gather(0)
                scale(gA, 0)
                wait_scatter(1)
                start_gather(gB, 1)
                start_scatter(gA, 0)
                wait_gather(1)
                scale(gB, 1)
                wait_scatter(0)
                start_gather(jnp.minimum(gB + 1, GBK - 1), 0)
                start_scatter(gB, 1)
                return pc
            lax.fori_loop(0, GBK // 2, pbody, 0)
            wait_gather(0)
            return carry

        start_scatter_trash()
        lax.fori_loop(0, nblk, blk, 0)
        wait_scatter(1)

        plsc.subcore_barrier()

        @pl.when(s == 0)
        def _flush():
            pltpu.sync_copy(out_sh.at[pl.ds(0, NV)], out_hbm.at[c])

    return k(srcg, dstg, pwg, pden, h)


def _tc_finish(outp, bias2):
    _, N, C = outp.shape

    def body(o_ref, b_ref, out_ref):
        out_ref[...] = o_ref[0] + o_ref[1] + b_ref[...]

    return pl.pallas_call(
        body,
        out_shape=jax.ShapeDtypeStruct((N, C), jnp.float32),
    )(outp, bias2)


def kernel(x, edge_index, edge_weight, modality_groups, W, att_src, att_dst, bias):
    N, D = x.shape
    C = W.shape[1]
    E = edge_index.shape[1]

    nm = jnp.zeros((N,), jnp.int32)
    for g in range(modality_groups.shape[0]):
        nm = nm.at[modality_groups[g]].set(g)

    QUANT = NW * GRP * GB
    EP = ((E + QUANT - 1) // QUANT) * QUANT
    pad = EP - E
    srcg = jnp.concatenate([edge_index[0], jnp.zeros((pad,), jnp.int32)]).reshape(EP // GRP, GRP)
    dstg = jnp.concatenate([edge_index[1], jnp.zeros((pad,), jnp.int32)]).reshape(EP // GRP, GRP)
    ewg = jnp.concatenate([edge_weight, jnp.zeros((pad,), jnp.float32)]).reshape(EP // GRP, GRP)

    asv = att_src.reshape(1, C).astype(jnp.float32)
    adv = att_dst.reshape(1, C).astype(jnp.float32)

    h, ab = _tc_project(x, W, asv, adv)
    pden, pwg = _sc_edge_pass(srcg, dstg, ewg, ab, nm)
    outp = _sc_msg_pass(srcg, dstg, pwg, pden, h)
    return _tc_finish(outp, bias.reshape(1, C))
